# Initial kernel scaffold; baseline (speedup 1.0000x reference)
import jax
import jax.numpy as jnp
from jax.experimental import pallas as pl


def kernel(input, table, cp):
    raise NotImplementedError("write your pallas kernel here")


# speedup vs baseline: 7.8474x; 7.8474x over previous
import functools

import jax
import jax.numpy as jnp
from jax import lax
from jax.experimental import pallas as pl
from jax.experimental.pallas import tpu as pltpu
from jax.experimental.pallas import tpu_sc as plsc

VOCAB = 100000
D = 128
P = 20
B = 4096
S = 200
T = S - P

NC = 2
NS = 16
NW = NC * NS
BPW = B // NW

_mesh = plsc.VectorSubcoreMesh(core_axis_name="c", subcore_axis_name="s")


@functools.partial(
    pl.kernel,
    mesh=_mesh,
    compiler_params=pltpu.CompilerParams(use_tc_tiling_on_sc=False),
    out_type=jax.ShapeDtypeStruct((B, S, D), jnp.float32),
    scratch_types=[
        pltpu.VMEM((184,), jnp.int32),
        pltpu.VMEM((S, D), jnp.float32),
        pltpu.SemaphoreType.DMA,
    ],
)
def _embed_kernel(inp_hbm, table_hbm, cp_hbm, out_hbm, idx_v, buf, sem):
    wid = lax.axis_index("s") * NC + lax.axis_index("c")
    base = wid * BPW

    pltpu.sync_copy(cp_hbm, buf.at[pl.ds(0, P)])

    def body(j, carry):
        b = base + j
        pltpu.sync_copy(inp_hbm.at[b, pl.ds(0, 184)], idx_v)
        g1 = pltpu.async_copy(
            table_hbm.at[idx_v.at[pl.ds(0, 128)]], buf.at[pl.ds(P, 128)], sem
        )
        g2 = pltpu.async_copy(
            table_hbm.at[idx_v.at[pl.ds(128, T - 128)]],
            buf.at[pl.ds(P + 128, T - 128)],
            sem,
        )
        g1.wait()
        g2.wait()
        pltpu.sync_copy(buf, out_hbm.at[b])
        return carry

    lax.fori_loop(0, BPW, body, 0)


def kernel(input, table, cp):
    return _embed_kernel(input.astype(jnp.int32), table, cp)


# speedup vs baseline: 11.5194x; 1.4679x over previous
import functools

import jax
import jax.numpy as jnp
from jax import lax
from jax.experimental import pallas as pl
from jax.experimental.pallas import tpu as pltpu
from jax.experimental.pallas import tpu_sc as plsc

VOCAB = 100000
D = 128
P = 20
B = 4096
S = 200
T = S - P
TP = 184

NC = 2
NS = 16
NW = NC * NS
BPW = B // NW

_mesh = plsc.VectorSubcoreMesh(core_axis_name="c", subcore_axis_name="s")


@functools.partial(
    pl.kernel,
    mesh=_mesh,
    compiler_params=pltpu.CompilerParams(use_tc_tiling_on_sc=False),
    out_type=jax.ShapeDtypeStruct((B, S, D), jnp.float32),
    scratch_types=[
        pltpu.VMEM((BPW, TP), jnp.int32),
        pltpu.VMEM((S + 4, D), jnp.float32),
        pltpu.VMEM((S + 4, D), jnp.float32),
        pltpu.SemaphoreType.DMA,
        pltpu.SemaphoreType.DMA,
        pltpu.SemaphoreType.DMA,
        pltpu.SemaphoreType.DMA,
    ],
)
def _embed_kernel(inp_hbm, table_hbm, cp_hbm, out_hbm,
                  idx_all, buf_a, buf_b, gsem_a, gsem_b, ssem_a, ssem_b):
    wid = lax.axis_index("s") * NC + lax.axis_index("c")
    base = wid * BPW

    pltpu.sync_copy(inp_hbm.at[pl.ds(base, BPW), pl.ds(0, TP)], idx_all)

    pltpu.sync_copy(cp_hbm, buf_a.at[pl.ds(0, P)])
    pltpu.sync_copy(cp_hbm, buf_b.at[pl.ds(0, P)])

    def one(j, buf, gsem, ssem):
        b = base + j

        @pl.when(j >= 2)
        def _():
            pltpu.make_async_copy(
                buf.at[pl.ds(0, S)], out_hbm.at[b], ssem).wait()

        g1 = pltpu.async_copy(
            table_hbm.at[idx_all.at[j, pl.ds(0, 128)]],
            buf.at[pl.ds(P, 128)], gsem,
        )
        g2 = pltpu.async_copy(
            table_hbm.at[idx_all.at[j, pl.ds(128, TP - 128)]],
            buf.at[pl.ds(P + 128, TP - 128)], gsem,
        )
        g1.wait()
        g2.wait()
        pltpu.async_copy(buf.at[pl.ds(0, S)], out_hbm.at[b], ssem)

    def body(i, carry):
        one(2 * i, buf_a, gsem_a, ssem_a)
        one(2 * i + 1, buf_b, gsem_b, ssem_b)
        return carry

    lax.fori_loop(0, BPW // 2, body, 0)

    pltpu.make_async_copy(
        buf_a.at[pl.ds(0, S)], out_hbm.at[base + BPW - 2], ssem_a).wait()
    pltpu.make_async_copy(
        buf_b.at[pl.ds(0, S)], out_hbm.at[base + BPW - 1], ssem_b).wait()


def kernel(input, table, cp):
    return _embed_kernel(input.astype(jnp.int32), table, cp)


# speedup vs baseline: 11.6808x; 1.0140x over previous
import functools

import jax
import jax.numpy as jnp
from jax import lax
from jax.experimental import pallas as pl
from jax.experimental.pallas import tpu as pltpu
from jax.experimental.pallas import tpu_sc as plsc

VOCAB = 100000
D = 128
P = 20
B = 4096
S = 200
T = S - P
TP = 184

NC = 2
NS = 16
NW = NC * NS
BPW = B // NW

NBUF = 3
AHEAD = 2

_mesh = plsc.VectorSubcoreMesh(core_axis_name="c", subcore_axis_name="s")


@functools.partial(
    pl.kernel,
    mesh=_mesh,
    compiler_params=pltpu.CompilerParams(use_tc_tiling_on_sc=False),
    out_type=jax.ShapeDtypeStruct((B, S, D), jnp.float32),
    scratch_types=[
        pltpu.VMEM((BPW, TP), jnp.int32),
        pltpu.VMEM((S + 4, D), jnp.float32),
        pltpu.VMEM((S + 4, D), jnp.float32),
        pltpu.VMEM((S + 4, D), jnp.float32),
        pltpu.SemaphoreType.DMA,
        pltpu.SemaphoreType.DMA,
        pltpu.SemaphoreType.DMA,
        pltpu.SemaphoreType.DMA,
        pltpu.SemaphoreType.DMA,
        pltpu.SemaphoreType.DMA,
    ],
)
def _embed_kernel(inp_hbm, table_hbm, cp_hbm, out_hbm, idx_all,
                  buf0, buf1, buf2, gsem0, gsem1, gsem2,
                  ssem0, ssem1, ssem2):
    bufs = (buf0, buf1, buf2)
    gsems = (gsem0, gsem1, gsem2)
    ssems = (ssem0, ssem1, ssem2)
    wid = lax.axis_index("s") * NC + lax.axis_index("c")
    base = wid * BPW

    pltpu.sync_copy(inp_hbm.at[pl.ds(base, BPW), pl.ds(0, TP)], idx_all)

    for buf in bufs:
        pltpu.sync_copy(cp_hbm, buf.at[pl.ds(0, P)])

    def fire_gathers(j, k):
        pltpu.async_copy(
            table_hbm.at[idx_all.at[j, pl.ds(0, 128)]],
            bufs[k].at[pl.ds(P, 128)], gsems[k],
        )
        pltpu.async_copy(
            table_hbm.at[idx_all.at[j, pl.ds(128, TP - 128)]],
            bufs[k].at[pl.ds(P + 128, TP - 128)], gsems[k],
        )

    def wait_gathers(j, k):
        pltpu.make_async_copy(
            table_hbm.at[idx_all.at[j, pl.ds(0, 128)]],
            bufs[k].at[pl.ds(P, 128)], gsems[k],
        ).wait()
        pltpu.make_async_copy(
            table_hbm.at[idx_all.at[j, pl.ds(128, TP - 128)]],
            bufs[k].at[pl.ds(P + 128, TP - 128)], gsems[k],
        ).wait()

    def wait_store(j, k):
        pltpu.make_async_copy(
            bufs[k].at[pl.ds(0, S)], out_hbm.at[base + j], ssems[k],
        ).wait()

    for j0 in range(AHEAD):
        fire_gathers(j0, j0 % NBUF)

    def body(i, carry):
        for k in range(NBUF):
            j = NBUF * i + k
            wait_gathers(j, k)
            pltpu.async_copy(
                bufs[k].at[pl.ds(0, S)], out_hbm.at[base + j], ssems[k],
            )
            jj = j + AHEAD
            kk = (k + AHEAD) % NBUF

            @pl.when(jj < BPW)
            def _():
                @pl.when(jj >= NBUF)
                def _():
                    wait_store(jj - NBUF, kk)

                fire_gathers(jj, kk)

        return carry

    lax.fori_loop(0, BPW // NBUF, body, 0)

    for r in range((BPW // NBUF) * NBUF, BPW):
        k = r % NBUF
        wait_gathers(r, k)
        pltpu.async_copy(
            bufs[k].at[pl.ds(0, S)], out_hbm.at[base + r], ssems[k],
        )

    for j in range(BPW - NBUF, BPW):
        wait_store(j, j % NBUF)


def kernel(input, table, cp):
    return _embed_kernel(input.astype(jnp.int32), table, cp)
